# baseline (speedup 1.0000x reference)
import functools
import jax
import jax.numpy as jnp
from jax import lax
from jax.experimental import pallas as pl
from jax.experimental.pallas import tpu as pltpu
from jax.experimental.pallas import tpu_sc as plsc

N_NODES_K = 10000
N_EDGES_K = 160000
D_K = 256
Q_K = D_K // 4
B_K = 128
NSUB = 16
L = 16
BLK_PER_SUB = 80
NBLK_PAD = BLK_PER_SUB * NSUB
E_PAD = NBLK_PAD * B_K
NBUF = 4
QUADS = BLK_PER_SUB // NBUF
ROWS_PER_SUB = 624

_mesh = plsc.VectorSubcoreMesh(core_axis_name="c", subcore_axis_name="s")


@functools.partial(
    pl.kernel,
    out_type=jax.ShapeDtypeStruct((4, N_NODES_K, Q_K), jnp.float32),
    mesh=_mesh,
    scratch_types=[
        pltpu.VMEM((BLK_PER_SUB, 2, B_K), jnp.int32),
        pltpu.VMEM((BLK_PER_SUB, B_K), jnp.float32),
        pltpu.VMEM((B_K, Q_K), jnp.float32),
        pltpu.VMEM((B_K, Q_K), jnp.float32),
        pltpu.VMEM((B_K, Q_K), jnp.float32),
        pltpu.VMEM((B_K, Q_K), jnp.float32),
        pltpu.VMEM_SHARED((N_NODES_K, Q_K), jnp.float32),
        pltpu.SemaphoreType.DMA,
        pltpu.SemaphoreType.DMA,
        pltpu.SemaphoreType.DMA,
        pltpu.SemaphoreType.DMA,
        pltpu.SemaphoreType.DMA,
        pltpu.SemaphoreType.DMA,
        pltpu.SemaphoreType.DMA,
        pltpu.SemaphoreType.DMA,
        pltpu.SemaphoreType.DMA,
    ],
    compiler_params=pltpu.CompilerParams(use_tc_tiling_on_sc=False),
)
def _spmm_sc(idx_h, val_h, o0_h, o1_h, o2_h, o3_h, out_h,
             idxb, valb, msg0, msg1, msg2, msg3, acc,
             sem_i, sg0, sg1, sg2, sg3, ss0, ss1, ss2, ss3):
    c = lax.axis_index("c")
    s = lax.axis_index("s")
    msgs = [msg0, msg1, msg2, msg3]
    sgs = [sg0, sg1, sg2, sg3]
    sss = [ss0, ss1, ss2, ss3]
    oqs = [o0_h, o1_h, o2_h, o3_h]
    zeros16 = jnp.zeros((L,), jnp.float32)
    r0 = s * ROWS_PER_SUB

    d1 = pltpu.async_copy(
        idx_h.at[pl.ds(s * BLK_PER_SUB, BLK_PER_SUB), :, :], idxb, sem_i)
    d2 = pltpu.async_copy(
        val_h.at[pl.ds(s * BLK_PER_SUB, BLK_PER_SUB), :], valb, sem_i)
    d1.wait()
    d2.wait()

    def gather_issue(t, j, p):
        @pl.when(c == 0)
        def _():
            pltpu.async_copy(oqs[2 * p].at[idxb.at[t, 1]], msgs[j], sgs[j])

        @pl.when(c == 1)
        def _():
            pltpu.async_copy(oqs[2 * p + 1].at[idxb.at[t, 1]], msgs[j], sgs[j])

    def gather_wait(j):
        pltpu.make_async_copy(
            o0_h.at[pl.ds(0, B_K), :], msgs[j], sgs[j]).wait()

    def scatter_drain(j):
        pltpu.make_async_copy(
            msgs[j], acc.at[pl.ds(0, B_K), :], sss[j]).wait()

    for p in range(2):
        @pl.loop(0, B_K)
        def _(r):
            for j in range(Q_K // L):
                msg0[r, pl.ds(j * L, L)] = zeros16

        for kk in range(4):
            pltpu.sync_copy(msg0, acc.at[pl.ds(r0 + kk * B_K, B_K), :])

        @pl.when(s < NSUB - 1)
        def _():
            pltpu.sync_copy(msg0.at[pl.ds(0, 112), :],
                            acc.at[pl.ds(r0 + 4 * B_K, 112), :])

        @pl.when(s == NSUB - 1)
        def _():
            pltpu.sync_copy(msg0, acc.at[pl.ds(r0 + 4 * B_K, B_K), :])

        plsc.subcore_barrier()

        gather_issue(0, 0, p)
        gather_issue(1, 1, p)

        @pl.loop(0, QUADS)
        def _(q):
            for j in range(NBUF):
                t = q * NBUF + j

                @pl.when(t >= 2)
                def _(j=j):
                    scatter_drain((j + 2) % NBUF)

                @pl.when(t + 2 < BLK_PER_SUB)
                def _(j=j, t=t):
                    gather_issue(t + 2, (j + 2) % NBUF, p)

                gather_wait(j)

                @pl.loop(0, B_K // L)
                def _(g, j=j, t=t):
                    vals16 = valb[t, pl.ds(g * L, L)]
                    for i in range(L):
                        vspl = jnp.take_along_axis(
                            vals16, jnp.full((L,), i, jnp.int32), axis=0)
                        e = g * L + i
                        for jj in range(Q_K // L):
                            sl = pl.ds(jj * L, L)
                            msgs[j][e, sl] = msgs[j][e, sl] * vspl

                pltpu.async_copy(msgs[j], acc.at[idxb.at[t, 0]], sss[j],
                                 add=True)

        scatter_drain(2)
        scatter_drain(3)
        plsc.subcore_barrier()

        for kk in range(4):
            pltpu.sync_copy(acc.at[pl.ds(r0 + kk * B_K, B_K), :],
                            out_h.at[2 * p + c, pl.ds(r0 + kk * B_K, B_K), :])

        @pl.when(s < NSUB - 1)
        def _():
            pltpu.sync_copy(acc.at[pl.ds(r0 + 4 * B_K, 112), :],
                            out_h.at[2 * p + c, pl.ds(r0 + 4 * B_K, 112), :])

        @pl.when(s == NSUB - 1)
        def _():
            pltpu.sync_copy(acc.at[pl.ds(r0 + 4 * B_K, B_K), :],
                            out_h.at[2 * p + c, pl.ds(r0 + 4 * B_K, B_K), :])

        plsc.subcore_barrier()


def kernel(row, col, value, other):
    pad = E_PAD - N_EDGES_K
    zi = jnp.zeros((pad,), jnp.int32)
    row2 = jnp.concatenate([row, zi]).reshape(NBLK_PAD, 1, B_K)
    col2 = jnp.concatenate([col, zi]).reshape(NBLK_PAD, 1, B_K)
    val2 = jnp.concatenate(
        [value, jnp.zeros((pad,), jnp.float32)]).reshape(NBLK_PAD, B_K)
    idxp = jnp.concatenate([row2, col2], axis=1)
    oq = [other[:, i * Q_K:(i + 1) * Q_K] for i in range(4)]
    out4 = _spmm_sc(idxp, val2, oq[0], oq[1], oq[2], oq[3])
    return out4.transpose(1, 0, 2).reshape(N_NODES_K, D_K)
